# baseline (speedup 1.0000x reference)
import functools

import jax
import jax.numpy as jnp
from jax.experimental import pallas as pl
from jax.experimental.pallas import tpu as pltpu

_INV_SQRT2 = 0.7071067811865476


def _adapter_block(x_ref, w1t_ref, w2t_ref, o_ref):
    x = x_ref[...]
    h = jnp.dot(x, w1t_ref[...], preferred_element_type=jnp.float32)
    h = 0.5 * h * (1.0 + jax.lax.erf(h * _INV_SQRT2))
    o_ref[...] = x + jnp.dot(h, w2t_ref[...], preferred_element_type=jnp.float32)


def _adapter_block2(x_ref, w1t_ref, w2t_ref, o_ref, h_ref):
    j = pl.program_id(1)

    @pl.when(j == 0)
    def _():
        h = jnp.dot(x_ref[...], w1t_ref[...], preferred_element_type=jnp.float32)
        h_ref[...] = 0.5 * h * (1.0 + jax.lax.erf(h * _INV_SQRT2))

    cols = o_ref.shape[1]
    o_ref[...] = x_ref[:, pl.ds(j * cols, cols)] + jnp.dot(
        h_ref[...], w2t_ref[...], preferred_element_type=jnp.float32)


@functools.partial(jax.jit, static_argnames=("block_rows", "col_splits"))
def _adapter2(x2d, w1t, w2t, block_rows, col_splits):
    n, d = x2d.shape
    m = w1t.shape[1]
    dc = d // col_splits
    out = pl.pallas_call(
        _adapter_block2,
        grid=(n // block_rows, col_splits),
        in_specs=[
            pl.BlockSpec((block_rows, d), lambda i, j: (i, 0)),
            pl.BlockSpec((d, m), lambda i, j: (0, 0)),
            pl.BlockSpec((m, dc), lambda i, j: (0, j)),
        ],
        out_specs=pl.BlockSpec((block_rows, dc), lambda i, j: (i, j)),
        out_shape=jax.ShapeDtypeStruct((n, d), jnp.float32),
        scratch_shapes=[pltpu.VMEM((block_rows, m), jnp.float32)],
        compiler_params=pltpu.CompilerParams(
            dimension_semantics=("arbitrary", "arbitrary"),
            vmem_limit_bytes=100 * 1024 * 1024,
        ),
    )(x2d, w1t, w2t)
    return out


@functools.partial(jax.jit, static_argnames=("block_rows",))
def _adapter(x2d, w1t, w2t, block_rows):
    n, d = x2d.shape
    m = w1t.shape[1]
    grid = (n // block_rows,)
    out = pl.pallas_call(
        _adapter_block,
        grid=grid,
        in_specs=[
            pl.BlockSpec((block_rows, d), lambda i: (i, 0)),
            pl.BlockSpec((d, m), lambda i: (0, 0)),
            pl.BlockSpec((m, d), lambda i: (0, 0)),
        ],
        out_specs=pl.BlockSpec((block_rows, d), lambda i: (i, 0)),
        out_shape=jax.ShapeDtypeStruct((n, d), jnp.float32),
        compiler_params=pltpu.CompilerParams(
            dimension_semantics=("parallel",),
            vmem_limit_bytes=100 * 1024 * 1024,
        ),
    )(x2d, w1t, w2t)
    return out


_BLOCK_ROWS = 2048


def kernel(x, W1, W2):
    b, s, d = x.shape
    n = b * s
    x2d = x.reshape(n, d)
    w1t, w2t = W1.T, W2.T

    ndev = jax.device_count()
    nsh = 1 << (max(ndev, 1).bit_length() - 1)
    while nsh > 1 and (n % (nsh * _BLOCK_ROWS)) != 0:
        nsh //= 2

    if nsh > 1:
        mesh = jax.make_mesh((nsh,), ("d",),
                             axis_types=(jax.sharding.AxisType.Auto,))
        P = jax.sharding.PartitionSpec
        x2d = jax.lax.with_sharding_constraint(
            x2d, jax.sharding.NamedSharding(mesh, P("d", None)))
        w1t = jax.lax.with_sharding_constraint(
            w1t, jax.sharding.NamedSharding(mesh, P(None, None)))
        w2t = jax.lax.with_sharding_constraint(
            w2t, jax.sharding.NamedSharding(mesh, P(None, None)))
        fn = jax.shard_map(
            lambda xs, a, c: _adapter(xs, a, c, _BLOCK_ROWS),
            mesh=mesh,
            in_specs=(P("d", None), P(None, None), P(None, None)),
            out_specs=P("d", None),
            check_vma=False,
        )
        out = fn(x2d, w1t, w2t)
    else:
        out = _adapter(x2d, w1t, w2t, _BLOCK_ROWS)
    return (out.reshape(b, s, d), jnp.float32(0.0))


# speedup vs baseline: 5.7219x; 5.7219x over previous
import functools

import jax
import jax.numpy as jnp
from jax.experimental import pallas as pl
from jax.experimental.pallas import tpu as pltpu

_INV_SQRT2 = 0.7071067811865476


def _adapter_block(x_ref, w1t_ref, w2t_ref, o_ref):
    x = x_ref[...]
    h = jnp.dot(x, w1t_ref[...], preferred_element_type=jnp.float32)
    h = 0.5 * h * (1.0 + jax.lax.erf(h * _INV_SQRT2))
    o_ref[...] = x + jnp.dot(h, w2t_ref[...], preferred_element_type=jnp.float32)


@functools.partial(jax.jit, static_argnames=("block_rows",))
def _adapter(x2d, w1t, w2t, block_rows):
    n, d = x2d.shape
    m = w1t.shape[1]
    grid = (n // block_rows,)
    out = pl.pallas_call(
        _adapter_block,
        grid=grid,
        in_specs=[
            pl.BlockSpec((block_rows, d), lambda i: (i, 0)),
            pl.BlockSpec((d, m), lambda i: (0, 0)),
            pl.BlockSpec((m, d), lambda i: (0, 0)),
        ],
        out_specs=pl.BlockSpec((block_rows, d), lambda i: (i, 0)),
        out_shape=jax.ShapeDtypeStruct((n, d), jnp.float32),
        compiler_params=pltpu.CompilerParams(
            dimension_semantics=("parallel",),
            vmem_limit_bytes=100 * 1024 * 1024,
        ),
    )(x2d, w1t, w2t)
    return out


_BLOCK_ROWS = 2048


def kernel(x, W1, W2):
    b, s, d = x.shape
    n = b * s
    x2d = x.reshape(n, d)
    out = _adapter(x2d, W1.T, W2.T, _BLOCK_ROWS)
    return (out.reshape(b, s, d), jnp.float32(0.0))
